# Initial kernel scaffold; baseline (speedup 1.0000x reference)
import jax
import jax.numpy as jnp
from jax.experimental import pallas as pl


def kernel(queries, keys, k):
    raise NotImplementedError("write your pallas kernel here")


# speedup vs baseline: 3.5200x; 3.5200x over previous
import jax
import jax.numpy as jnp
from jax.experimental import pallas as pl
from jax.experimental.pallas import tpu as pltpu

_KB = 1000


def _nn_kernel(q32_ref, q16_ref, k_ref, out_ref, acc_ref):
    i = pl.program_id(0)
    nk = pl.num_programs(0)

    kblk = k_ref[...]
    ones_row = jnp.ones((1, kblk.shape[1]), jnp.float32)
    k_sq = jax.lax.dot_general(
        ones_row, kblk * kblk,
        (((1,), (1,)), ((), ())),
        preferred_element_type=jnp.float32)
    dot = jax.lax.dot_general(
        q16_ref[...], kblk.astype(jnp.bfloat16),
        (((1,), (1,)), ((), ())),
        preferred_element_type=jnp.float32)
    part = jnp.min(k_sq - 2.0 * dot, axis=1, keepdims=True)

    @pl.when(i == 0)
    def _init():
        acc_ref[...] = part

    @pl.when(i > 0)
    def _update():
        acc_ref[...] = jnp.minimum(acc_ref[...], part)

    @pl.when(i == nk - 1)
    def _finalize():
        q = q32_ref[...]
        q_sq = jnp.sum(q * q, axis=1, keepdims=True)
        out_ref[...] = jnp.sqrt(jnp.maximum(q_sq + acc_ref[...], 1e-12))


def kernel(queries, keys, k):
    Q, D = queries.shape
    K = keys.shape[0]
    nk = K // _KB
    q16 = queries.astype(jnp.bfloat16)
    out = pl.pallas_call(
        _nn_kernel,
        grid=(nk,),
        in_specs=[
            pl.BlockSpec((Q, D), lambda i: (0, 0)),
            pl.BlockSpec((Q, D), lambda i: (0, 0)),
            pl.BlockSpec((_KB, D), lambda i: (i, 0)),
        ],
        out_specs=pl.BlockSpec((Q, 1), lambda i: (0, 0)),
        out_shape=jax.ShapeDtypeStruct((Q, 1), jnp.float32),
        scratch_shapes=[pltpu.VMEM((Q, 1), jnp.float32)],
    )(queries, q16, keys)
    return (out[:, 0] / k).astype(jnp.float32)


# speedup vs baseline: 3.7293x; 1.0595x over previous
import jax
import jax.numpy as jnp
from jax.experimental import pallas as pl

_KB = 2000
_W = 512


def _nn_kernel(qm2_ref, k_ref, acc_ref):
    i = pl.program_id(0)

    @pl.when(i == 0)
    def _init():
        acc_ref[...] = jnp.full(acc_ref.shape, jnp.inf, jnp.float32)

    kb16 = k_ref[...].astype(jnp.bfloat16)
    sq16 = kb16 * kb16
    ones_row = jnp.ones((1, sq16.shape[1]), jnp.bfloat16)
    ksq = jax.lax.dot_general(
        ones_row, sq16, (((1,), (1,)), ((), ())),
        preferred_element_type=jnp.float32)
    qm2 = qm2_ref[...]
    for s in range(0, _KB, _W):
        w = min(_W, _KB - s)
        dotj = jax.lax.dot_general(
            qm2, kb16[s:s + w, :], (((1,), (1,)), ((), ())),
            preferred_element_type=jnp.float32)
        tmp = dotj + ksq[:, s:s + w]
        acc_ref[:, 0:w] = jnp.minimum(acc_ref[:, 0:w], tmp)


def _fin_kernel(q_ref, acc_ref, out_ref):
    q = q_ref[...]
    q_sq = jnp.sum(q * q, axis=1, keepdims=True)
    m = jnp.min(acc_ref[...], axis=1, keepdims=True)
    out_ref[...] = jnp.sqrt(jnp.maximum(q_sq + m, 1e-12))


def kernel(queries, keys, k):
    Q, D = queries.shape
    K = keys.shape[0]
    nk = K // _KB
    qm2 = (queries * -2.0).astype(jnp.bfloat16)
    acc = pl.pallas_call(
        _nn_kernel,
        grid=(nk,),
        in_specs=[
            pl.BlockSpec((Q, D), lambda i: (0, 0)),
            pl.BlockSpec((_KB, D), lambda i: (i, 0)),
        ],
        out_specs=pl.BlockSpec((Q, _W), lambda i: (0, 0)),
        out_shape=jax.ShapeDtypeStruct((Q, _W), jnp.float32),
    )(qm2, keys)
    out = pl.pallas_call(
        _fin_kernel,
        in_specs=[
            pl.BlockSpec((Q, D), lambda i: (0, 0)),
            pl.BlockSpec((Q, _W), lambda i: (0, 0)),
        ],
        out_specs=pl.BlockSpec((Q, 1), lambda i: (0, 0)),
        out_shape=jax.ShapeDtypeStruct((Q, 1), jnp.float32),
        grid=(1,),
    )(queries, acc)
    return (out[:, 0] / k).astype(jnp.float32)
